# baseline (speedup 1.0000x reference)
import functools

import jax
import jax.numpy as jnp
from jax.experimental import pallas as pl
from jax.experimental.pallas import tpu as pltpu

_G = 512


@functools.cache
def _build(B, V, D):
    n_steps = B // _G

    def tck(idx_smem, table_any, out_vmem, buf, sems):
        i = pl.program_id(0)

        @pl.when(i < n_steps)
        def _issue():
            for g in range(_G):
                r = idx_smem[i * _G + g]
                pltpu.make_async_copy(
                    table_any.at[pl.ds(r, 1)],
                    buf.at[i % 2, pl.ds(g, 1)],
                    sems.at[i % 2],
                ).start()

        @pl.when(i > 0)
        def _drain():
            pltpu.make_async_copy(
                table_any.at[pl.ds(0, _G)],
                buf.at[(i - 1) % 2],
                sems.at[(i - 1) % 2],
            ).wait()
            out_vmem[...] = buf[(i - 1) % 2]

    return pl.pallas_call(
        tck,
        grid_spec=pltpu.PrefetchScalarGridSpec(
            num_scalar_prefetch=1,
            grid=(n_steps + 1,),
            in_specs=[pl.BlockSpec(memory_space=pl.ANY)],
            out_specs=pl.BlockSpec(
                (_G, D), lambda i, idx: (jnp.maximum(i - 1, 0), 0)
            ),
            scratch_shapes=[
                pltpu.VMEM((2, _G, D), jnp.float32),
                pltpu.SemaphoreType.DMA((2,)),
            ],
        ),
        out_shape=jax.ShapeDtypeStruct((B, D), jnp.float32),
    )


def kernel(label, table):
    (B,) = label.shape
    V, D = table.shape
    tc = _build(B, V, D)
    return tc(label.astype(jnp.int32), table)


# speedup vs baseline: 1.1150x; 1.1150x over previous
import functools

import jax
import jax.numpy as jnp
from jax import lax
from jax.experimental import pallas as pl
from jax.experimental.pallas import tpu as pltpu
from jax.experimental.pallas import tpu_sc as plsc


@functools.cache
def _build(B, V, D):
    info = plsc.get_sparse_core_info()
    nc, ns = info.num_cores, info.num_subcores
    nw = nc * ns
    b_per_w = B // nw
    n_groups = b_per_w // 16
    mesh = plsc.VectorSubcoreMesh(core_axis_name="c", subcore_axis_name="s")

    @functools.partial(
        pl.kernel,
        mesh=mesh,
        out_type=jax.ShapeDtypeStruct((B, D), jnp.float32),
        compiler_params=pltpu.CompilerParams(needs_layout_passes=False),
        scratch_types=[
            pltpu.VMEM((n_groups, 16), jnp.int32),
            pltpu.VMEM((b_per_w, D), jnp.float32),
            pltpu.SemaphoreType.DMA,
        ],
    )
    def emb(table_hbm, idx_hbm, out_hbm, idx_v, rows_v, sem):
        wid = lax.axis_index("s") * nc + lax.axis_index("c")
        pltpu.sync_copy(idx_hbm.at[wid], idx_v)
        lanes = lax.iota(jnp.int32, 16)

        @plsc.parallel_loop(0, n_groups, 1, unroll=2)
        def body(g):
            vec = idx_v[g, :]
            for l in range(16):
                r = jnp.sum(jnp.where(lanes == l, vec, 0))
                pltpu.async_copy(
                    table_hbm.at[r], rows_v.at[g * 16 + l], sem
                )

        pltpu.make_async_copy(
            table_hbm.at[pl.ds(0, b_per_w)], rows_v, sem
        ).wait()
        pltpu.sync_copy(
            rows_v.reshape(b_per_w // 8, 8, D),
            out_hbm.reshape(B // 8, 8, D).at[
                pl.ds(wid * (b_per_w // 8), b_per_w // 8)
            ],
        )

    return emb, nw, n_groups


def kernel(label, table):
    (B,) = label.shape
    V, D = table.shape
    emb, nw, n_groups = _build(B, V, D)
    idx = label.astype(jnp.int32).reshape(nw, n_groups, 16)
    return emb(table, idx)
